# Initial kernel scaffold; baseline (speedup 1.0000x reference)
import jax
import jax.numpy as jnp
from jax.experimental import pallas as pl


def kernel(e_emb, b_emb, s_emb, edge_index, b_curvature, s_curvature, eW, eb, bW, bb, sW, sb):
    raise NotImplementedError("write your pallas kernel here")


# speedup vs baseline: 6.9040x; 6.9040x over previous
import jax
import jax.numpy as jnp
from jax import lax
from jax.experimental import pallas as pl
from jax.experimental.pallas import tpu as pltpu
from jax.experimental.pallas import tpu_sc as plsc

_NC = 2
_NS = 16
_HF = 64
_DW = 16
_R = 2000
_ZR = 128
_NP = 10240


def _leaky(x):
    return jnp.where(x >= 0.0, x, 0.2 * x)


def _l2n(x):
    n = jnp.sqrt(jnp.sum(x * x, axis=-1, keepdims=True))
    return x / jnp.maximum(n, 1e-12)


def _logmap0(b, sc):
    n = jnp.maximum(jnp.sqrt(jnp.sum(b * b, axis=-1, keepdims=True)), 1e-10)
    z = sc * n
    atanh = 0.5 * jnp.log((1.0 + z) / (1.0 - z))
    return (2.0 / sc) * atanh * b / n


def _expmap0(v, sc):
    n = jnp.maximum(jnp.sqrt(jnp.sum(v * v, axis=-1, keepdims=True)), 1e-10)
    return jnp.tanh(sc * n * 0.5) * v / (sc * n)


def _fuse_pre(e, b, s, ewt, eb1, bwt, bb1, swt, sb1, sc):
    he = jnp.dot(e, ewt, preferred_element_type=jnp.float32) + eb1
    hb = jnp.dot(_logmap0(b, sc), bwt, preferred_element_type=jnp.float32) + bb1
    hs = _l2n(jnp.dot(_l2n(s), swt, preferred_element_type=jnp.float32) + sb1)
    return he, jnp.concatenate([hb, hs], axis=1)


def _agg_split(plo_ref, phi_ref, d_ref, sc):
    deg = jnp.maximum(d_ref[...][:, :1], 1.0)
    e1 = _leaky(plo_ref[...] / deg)
    phi = phi_ref[...] / deg
    b1 = _expmap0(phi[:, :32], sc)
    s1 = _l2n(phi[:, 32:])
    return e1, b1, s1


def _wspecs(ed, bd, sd):
    return [
        pl.BlockSpec((ed, ed), lambda i: (0, 0)),
        pl.BlockSpec((1, ed), lambda i: (0, 0)),
        pl.BlockSpec((bd, bd), lambda i: (0, 0)),
        pl.BlockSpec((1, bd), lambda i: (0, 0)),
        pl.BlockSpec((sd, sd), lambda i: (0, 0)),
        pl.BlockSpec((1, sd), lambda i: (0, 0)),
    ]


def _tc_pre(c2, e_emb, b_emb, s_emb, w):
    n, ed = e_emb.shape
    bd = b_emb.shape[1]
    sd = s_emb.shape[1]

    def body(c_ref, e_ref, b_ref, s_ref, ewt, eb1, bwt, bb1, swt, sb1,
             hlo_ref, hhi_ref):
        sc = jnp.sqrt(c_ref[0, 0])
        hlo, hhi = _fuse_pre(e_ref[...], b_ref[...], s_ref[...],
                             ewt[...], eb1[...], bwt[...], bb1[...],
                             swt[...], sb1[...], sc)
        hlo_ref[...] = hlo
        hhi_ref[...] = hhi

    return pl.pallas_call(
        body,
        grid=(n // _R,),
        in_specs=[
            pl.BlockSpec(memory_space=pltpu.SMEM),
            pl.BlockSpec((_R, ed), lambda i: (i, 0)),
            pl.BlockSpec((_R, bd), lambda i: (i, 0)),
            pl.BlockSpec((_R, sd), lambda i: (i, 0)),
        ] + _wspecs(ed, bd, sd),
        out_specs=[
            pl.BlockSpec((_R, _HF), lambda i: (i, 0)),
            pl.BlockSpec((_R, _HF), lambda i: (i, 0)),
        ],
        out_shape=[
            jax.ShapeDtypeStruct((_NP, _HF), jnp.float32),
            jax.ShapeDtypeStruct((_NP, _HF), jnp.float32),
        ],
    )(c2, e_emb, b_emb, s_emb, *w)


def _tc_mid(c2, plo, phi, d, w, n, ed, bd, sd):
    def body(c_ref, plo_ref, phi_ref, d_ref, ewt, eb1, bwt, bb1, swt, sb1,
             hlo_ref, hhi_ref):
        sc = jnp.sqrt(c_ref[0, 0])
        e1, b1, s1 = _agg_split(plo_ref, phi_ref, d_ref, sc)
        hlo, hhi = _fuse_pre(e1, b1, s1, ewt[...], eb1[...], bwt[...],
                             bb1[...], swt[...], sb1[...], sc)
        hlo_ref[...] = hlo
        hhi_ref[...] = hhi

    return pl.pallas_call(
        body,
        grid=(n // _R,),
        in_specs=[
            pl.BlockSpec(memory_space=pltpu.SMEM),
            pl.BlockSpec((_R, _HF), lambda i: (i, 0)),
            pl.BlockSpec((_R, _HF), lambda i: (i, 0)),
            pl.BlockSpec((_R, _DW), lambda i: (i, 0)),
        ] + _wspecs(ed, bd, sd),
        out_specs=[
            pl.BlockSpec((_R, _HF), lambda i: (i, 0)),
            pl.BlockSpec((_R, _HF), lambda i: (i, 0)),
        ],
        out_shape=[
            jax.ShapeDtypeStruct((_NP, _HF), jnp.float32),
            jax.ShapeDtypeStruct((_NP, _HF), jnp.float32),
        ],
    )(c2, plo, phi, d, *w)


def _tc_post(c2, plo, phi, d, n, ed, bd, sd):
    def body(c_ref, plo_ref, phi_ref, d_ref, e_ref, b_ref, s_ref):
        sc = jnp.sqrt(c_ref[0, 0])
        e1, b1, s1 = _agg_split(plo_ref, phi_ref, d_ref, sc)
        e_ref[...] = e1
        b_ref[...] = b1
        s_ref[...] = s1

    return pl.pallas_call(
        body,
        grid=(n // _R,),
        in_specs=[
            pl.BlockSpec(memory_space=pltpu.SMEM),
            pl.BlockSpec((_R, _HF), lambda i: (i, 0)),
            pl.BlockSpec((_R, _HF), lambda i: (i, 0)),
            pl.BlockSpec((_R, _DW), lambda i: (i, 0)),
        ],
        out_specs=[
            pl.BlockSpec((_R, ed), lambda i: (i, 0)),
            pl.BlockSpec((_R, bd), lambda i: (i, 0)),
            pl.BlockSpec((_R, sd), lambda i: (i, 0)),
        ],
        out_shape=[
            jax.ShapeDtypeStruct((n, ed), jnp.float32),
            jax.ShapeDtypeStruct((n, bd), jnp.float32),
            jax.ShapeDtypeStruct((n, sd), jnp.float32),
        ],
    )(c2, plo, phi, d)


def _sc_agg(hlo, hhi, src3, dst3, with_deg):
    n = hlo.shape[0]
    nch, cb = src3.shape[1], src3.shape[2]
    rpt = n // _NS

    mesh = plsc.VectorSubcoreMesh(core_axis_name="c", subcore_axis_name="s",
                                  num_cores=_NC, num_subcores=_NS)
    out_type = [jax.ShapeDtypeStruct((n, _HF), jnp.float32),
                jax.ShapeDtypeStruct((n, _HF), jnp.float32)]
    scratch = [
        pltpu.VMEM((nch, cb), jnp.int32),
        pltpu.VMEM((nch, cb), jnp.int32),
        pltpu.VMEM((cb, _HF), jnp.float32),
        pltpu.VMEM((_ZR, _HF), jnp.float32),
        pltpu.VMEM_SHARED((n, _HF), jnp.float32),
    ]
    if with_deg:
        out_type.append(jax.ShapeDtypeStruct((n, _DW), jnp.float32))
        scratch += [
            pltpu.VMEM((cb, _DW), jnp.float32),
            pltpu.VMEM((rpt, _DW), jnp.float32),
            pltpu.VMEM_SHARED((n, _DW), jnp.float32),
        ]

    def body(hlo_hbm, hhi_hbm, src_hbm, dst_hbm, *rest):
        if with_deg:
            plo_out, phi_out, d_out, srcv, dstv, rows, zbuf, acc, \
                ones, zdbuf, dacc = rest
        else:
            plo_out, phi_out, srcv, dstv, rows, zbuf, acc = rest
        cid = lax.axis_index("c")
        sid = lax.axis_index("s")
        base = sid * rpt
        z16 = jnp.zeros((16,), jnp.float32)

        def _zrow(r, carry):
            for k in range(_HF // 16):
                zbuf[r, pl.ds(k * 16, 16)] = z16
            return carry
        lax.fori_loop(0, _ZR, _zrow, 0)
        for t in range(rpt // _ZR):
            pltpu.sync_copy(zbuf, acc.at[pl.ds(base + t * _ZR, _ZR)])
        if with_deg:
            one16 = jnp.ones((16,), jnp.float32)

            def _drow(r, carry):
                zdbuf[r] = z16
                return carry
            lax.fori_loop(0, rpt, _drow, 0)
            pltpu.sync_copy(zdbuf, dacc.at[pl.ds(base, rpt)])

            def _orow(r, carry):
                ones[r] = one16
                return carry
            lax.fori_loop(0, cb, _orow, 0)
        pltpu.sync_copy(src_hbm.at[sid], srcv)
        pltpu.sync_copy(dst_hbm.at[sid], dstv)
        plsc.subcore_barrier()

        def _chunk(j, carry):
            @pl.when(cid == 0)
            def _():
                pltpu.sync_copy(hlo_hbm.at[srcv.at[j]], rows)

            @pl.when(cid == 1)
            def _():
                pltpu.sync_copy(hhi_hbm.at[srcv.at[j]], rows)
            pltpu.sync_copy(rows, acc.at[dstv.at[j]], add=True)
            if with_deg:
                @pl.when(cid == 1)
                def _():
                    pltpu.sync_copy(ones, dacc.at[dstv.at[j]], add=True)
            return carry
        lax.fori_loop(0, nch, _chunk, 0)
        plsc.subcore_barrier()

        for t in range(rpt // _ZR):
            sl = pl.ds(base + t * _ZR, _ZR)
            pltpu.sync_copy(acc.at[sl], zbuf)

            @pl.when(cid == 0)
            def _():
                pltpu.sync_copy(zbuf, plo_out.at[sl])

            @pl.when(cid == 1)
            def _():
                pltpu.sync_copy(zbuf, phi_out.at[sl])
        if with_deg:
            @pl.when(cid == 1)
            def _():
                pltpu.sync_copy(dacc.at[pl.ds(base, rpt)], zdbuf)
                pltpu.sync_copy(zdbuf, d_out.at[pl.ds(base, rpt)])

    f = pl.kernel(body, out_type=out_type, mesh=mesh, scratch_types=scratch,
                  compiler_params=pltpu.CompilerParams(use_tc_tiling_on_sc=False))
    return f(hlo, hhi, src3, dst3)


def kernel(e_emb, b_emb, s_emb, edge_index, b_curvature, s_curvature,
           eW, eb, bW, bb, sW, sb):
    n, ed = e_emb.shape
    bd = b_emb.shape[1]
    sd = s_emb.shape[1]
    e_edges = edge_index.shape[1]
    per_t = e_edges // _NS
    cb = 80
    nch = per_t // cb
    src3 = edge_index[0].reshape(_NS, nch, cb)
    dst3 = edge_index[1].reshape(_NS, nch, cb)
    c2 = b_curvature.reshape(1, 1)

    def wlayer(l):
        return (eW[l].T, eb[l].reshape(1, ed), bW[l].T, bb[l].reshape(1, bd),
                sW[l].T, sb[l].reshape(1, sd))

    hlo0, hhi0 = _tc_pre(c2, e_emb, b_emb, s_emb, wlayer(0))
    plo0, phi0, d0 = _sc_agg(hlo0, hhi0, src3, dst3, with_deg=True)
    hlo1, hhi1 = _tc_mid(c2, plo0, phi0, d0, wlayer(1), n, ed, bd, sd)
    plo1, phi1 = _sc_agg(hlo1, hhi1, src3, dst3, with_deg=False)
    e2, b2, s2 = _tc_post(c2, plo1, phi1, d0, n, ed, bd, sd)
    return (e2, b2, s2)


# speedup vs baseline: 12.3495x; 1.7887x over previous
import jax
import jax.numpy as jnp
from jax import lax
from jax.experimental import pallas as pl
from jax.experimental.pallas import tpu as pltpu
from jax.experimental.pallas import tpu_sc as plsc

_NC = 2
_NS = 16
_HF = 64
_DW = 16
_R = 2000
_ZR = 128
_NP = 10240


def _leaky(x):
    return jnp.where(x >= 0.0, x, 0.2 * x)


def _l2n(x):
    n = jnp.sqrt(jnp.sum(x * x, axis=-1, keepdims=True))
    return x / jnp.maximum(n, 1e-12)


def _logmap0(b, sc):
    n = jnp.maximum(jnp.sqrt(jnp.sum(b * b, axis=-1, keepdims=True)), 1e-10)
    z = sc * n
    atanh = 0.5 * jnp.log((1.0 + z) / (1.0 - z))
    return (2.0 / sc) * atanh * b / n


def _expmap0(v, sc):
    n = jnp.maximum(jnp.sqrt(jnp.sum(v * v, axis=-1, keepdims=True)), 1e-10)
    return jnp.tanh(sc * n * 0.5) * v / (sc * n)


def _fuse_pre(e, b, s, ewt, eb1, bwt, bb1, swt, sb1, sc):
    he = jnp.dot(e, ewt, preferred_element_type=jnp.float32) + eb1
    hb = jnp.dot(_logmap0(b, sc), bwt, preferred_element_type=jnp.float32) + bb1
    hs = _l2n(jnp.dot(_l2n(s), swt, preferred_element_type=jnp.float32) + sb1)
    return he, jnp.concatenate([hb, hs], axis=1)


def _agg_split(plo_ref, phi_ref, d_ref, sc):
    deg = jnp.maximum(d_ref[...][:, :1], 1.0)
    e1 = _leaky(plo_ref[...] / deg)
    phi = phi_ref[...] / deg
    b1 = _expmap0(phi[:, :32], sc)
    s1 = _l2n(phi[:, 32:])
    return e1, b1, s1


def _wspecs(ed, bd, sd):
    return [
        pl.BlockSpec((ed, ed), lambda i: (0, 0)),
        pl.BlockSpec((1, ed), lambda i: (0, 0)),
        pl.BlockSpec((bd, bd), lambda i: (0, 0)),
        pl.BlockSpec((1, bd), lambda i: (0, 0)),
        pl.BlockSpec((sd, sd), lambda i: (0, 0)),
        pl.BlockSpec((1, sd), lambda i: (0, 0)),
    ]


def _tc_pre(c2, e_emb, b_emb, s_emb, w):
    n, ed = e_emb.shape
    bd = b_emb.shape[1]
    sd = s_emb.shape[1]

    def body(c_ref, e_ref, b_ref, s_ref, ewt, eb1, bwt, bb1, swt, sb1,
             hlo_ref, hhi_ref):
        sc = jnp.sqrt(c_ref[0, 0])
        hlo, hhi = _fuse_pre(e_ref[...], b_ref[...], s_ref[...],
                             ewt[...], eb1[...], bwt[...], bb1[...],
                             swt[...], sb1[...], sc)
        hlo_ref[...] = hlo
        hhi_ref[...] = hhi

    return pl.pallas_call(
        body,
        grid=(n // _R,),
        in_specs=[
            pl.BlockSpec(memory_space=pltpu.SMEM),
            pl.BlockSpec((_R, ed), lambda i: (i, 0)),
            pl.BlockSpec((_R, bd), lambda i: (i, 0)),
            pl.BlockSpec((_R, sd), lambda i: (i, 0)),
        ] + _wspecs(ed, bd, sd),
        out_specs=[
            pl.BlockSpec((_R, _HF), lambda i: (i, 0)),
            pl.BlockSpec((_R, _HF), lambda i: (i, 0)),
        ],
        out_shape=[
            jax.ShapeDtypeStruct((_NP, _HF), jnp.float32),
            jax.ShapeDtypeStruct((_NP, _HF), jnp.float32),
        ],
    )(c2, e_emb, b_emb, s_emb, *w)


def _tc_mid(c2, plo, phi, d, w, n, ed, bd, sd):
    def body(c_ref, plo_ref, phi_ref, d_ref, ewt, eb1, bwt, bb1, swt, sb1,
             hlo_ref, hhi_ref):
        sc = jnp.sqrt(c_ref[0, 0])
        e1, b1, s1 = _agg_split(plo_ref, phi_ref, d_ref, sc)
        hlo, hhi = _fuse_pre(e1, b1, s1, ewt[...], eb1[...], bwt[...],
                             bb1[...], swt[...], sb1[...], sc)
        hlo_ref[...] = hlo
        hhi_ref[...] = hhi

    return pl.pallas_call(
        body,
        grid=(n // _R,),
        in_specs=[
            pl.BlockSpec(memory_space=pltpu.SMEM),
            pl.BlockSpec((_R, _HF), lambda i: (i, 0)),
            pl.BlockSpec((_R, _HF), lambda i: (i, 0)),
            pl.BlockSpec((_R, _DW), lambda i: (i, 0)),
        ] + _wspecs(ed, bd, sd),
        out_specs=[
            pl.BlockSpec((_R, _HF), lambda i: (i, 0)),
            pl.BlockSpec((_R, _HF), lambda i: (i, 0)),
        ],
        out_shape=[
            jax.ShapeDtypeStruct((_NP, _HF), jnp.float32),
            jax.ShapeDtypeStruct((_NP, _HF), jnp.float32),
        ],
    )(c2, plo, phi, d, *w)


def _tc_post(c2, plo, phi, d, n, ed, bd, sd):
    def body(c_ref, plo_ref, phi_ref, d_ref, e_ref, b_ref, s_ref):
        sc = jnp.sqrt(c_ref[0, 0])
        e1, b1, s1 = _agg_split(plo_ref, phi_ref, d_ref, sc)
        e_ref[...] = e1
        b_ref[...] = b1
        s_ref[...] = s1

    return pl.pallas_call(
        body,
        grid=(n // _R,),
        in_specs=[
            pl.BlockSpec(memory_space=pltpu.SMEM),
            pl.BlockSpec((_R, _HF), lambda i: (i, 0)),
            pl.BlockSpec((_R, _HF), lambda i: (i, 0)),
            pl.BlockSpec((_R, _DW), lambda i: (i, 0)),
        ],
        out_specs=[
            pl.BlockSpec((_R, ed), lambda i: (i, 0)),
            pl.BlockSpec((_R, bd), lambda i: (i, 0)),
            pl.BlockSpec((_R, sd), lambda i: (i, 0)),
        ],
        out_shape=[
            jax.ShapeDtypeStruct((n, ed), jnp.float32),
            jax.ShapeDtypeStruct((n, bd), jnp.float32),
            jax.ShapeDtypeStruct((n, sd), jnp.float32),
        ],
    )(c2, plo, phi, d)


def _sc_agg(hlo, hhi, src3, dst3, with_deg):
    n = hlo.shape[0]
    nch, cb = src3.shape[1], src3.shape[2]
    rpt = n // _NS

    mesh = plsc.VectorSubcoreMesh(core_axis_name="c", subcore_axis_name="s",
                                  num_cores=_NC, num_subcores=_NS)
    out_type = [jax.ShapeDtypeStruct((n, _HF), jnp.float32),
                jax.ShapeDtypeStruct((n, _HF), jnp.float32)]
    scratch = [
        pltpu.VMEM((nch, cb), jnp.int32),
        pltpu.VMEM((nch, cb), jnp.int32),
        pltpu.VMEM((2, cb, _HF), jnp.float32),
        pltpu.VMEM((_ZR, _HF), jnp.float32),
        pltpu.VMEM_SHARED((n, _HF), jnp.float32),
        pltpu.SemaphoreType.DMA,
        pltpu.SemaphoreType.DMA,
    ]
    if with_deg:
        out_type.append(jax.ShapeDtypeStruct((n, _DW), jnp.float32))
        scratch += [
            pltpu.VMEM((cb, _DW), jnp.float32),
            pltpu.VMEM((rpt, _DW), jnp.float32),
            pltpu.VMEM_SHARED((n, _DW), jnp.float32),
        ]

    def body(hlo_hbm, hhi_hbm, src_hbm, dst_hbm, *rest):
        if with_deg:
            plo_out, phi_out, d_out, srcv, dstv, rows, zbuf, acc, \
                gsem0, gsem1, ones, zdbuf, dacc = rest
        else:
            plo_out, phi_out, srcv, dstv, rows, zbuf, acc, gsem0, gsem1 = rest
        cid = lax.axis_index("c")
        sid = lax.axis_index("s")
        base = sid * rpt
        z16 = jnp.zeros((16,), jnp.float32)

        def _zrow(r, carry):
            for k in range(_HF // 16):
                zbuf[r, pl.ds(k * 16, 16)] = z16
            return carry
        lax.fori_loop(0, _ZR, _zrow, 0)
        for t in range(rpt // _ZR):
            pltpu.sync_copy(zbuf, acc.at[pl.ds(base + t * _ZR, _ZR)])
        if with_deg:
            one16 = jnp.ones((16,), jnp.float32)

            def _drow(r, carry):
                zdbuf[r] = z16
                return carry
            lax.fori_loop(0, rpt, _drow, 0)
            pltpu.sync_copy(zdbuf, dacc.at[pl.ds(base, rpt)])

            def _orow(r, carry):
                ones[r] = one16
                return carry
            lax.fori_loop(0, cb, _orow, 0)
        pltpu.sync_copy(src_hbm.at[sid], srcv)
        pltpu.sync_copy(dst_hbm.at[sid], dstv)
        plsc.subcore_barrier()

        def _gather(j, b, sem):
            @pl.when(cid == 0)
            def _():
                pltpu.async_copy(hlo_hbm.at[srcv.at[j]], rows.at[b], sem)

            @pl.when(cid == 1)
            def _():
                pltpu.async_copy(hhi_hbm.at[srcv.at[j]], rows.at[b], sem)

        def _gwait(j, b, sem):
            pltpu.make_async_copy(hlo_hbm.at[srcv.at[j]], rows.at[b], sem).wait()

        def _scat(j, b):
            pltpu.sync_copy(rows.at[b], acc.at[dstv.at[j]], add=True)
            if with_deg:
                @pl.when(cid == 1)
                def _():
                    pltpu.sync_copy(ones, dacc.at[dstv.at[j]], add=True)

        nloop = nch // 2
        _gather(0, 0, gsem0)

        def _pair(g, carry):
            j0 = 2 * g
            _gather(j0 + 1, 1, gsem1)
            _gwait(j0, 0, gsem0)
            _scat(j0, 0)

            @pl.when(g + 1 < nloop)
            def _():
                _gather(j0 + 2, 0, gsem0)
            _gwait(j0 + 1, 1, gsem1)
            _scat(j0 + 1, 1)
            return carry
        lax.fori_loop(0, nloop, _pair, 0)
        plsc.subcore_barrier()

        for t in range(rpt // _ZR):
            sl = pl.ds(base + t * _ZR, _ZR)
            pltpu.sync_copy(acc.at[sl], zbuf)

            @pl.when(cid == 0)
            def _():
                pltpu.sync_copy(zbuf, plo_out.at[sl])

            @pl.when(cid == 1)
            def _():
                pltpu.sync_copy(zbuf, phi_out.at[sl])
        if with_deg:
            @pl.when(cid == 1)
            def _():
                pltpu.sync_copy(dacc.at[pl.ds(base, rpt)], zdbuf)
                pltpu.sync_copy(zdbuf, d_out.at[pl.ds(base, rpt)])

    f = pl.kernel(body, out_type=out_type, mesh=mesh, scratch_types=scratch,
                  compiler_params=pltpu.CompilerParams(use_tc_tiling_on_sc=False))
    return f(hlo, hhi, src3, dst3)


def kernel(e_emb, b_emb, s_emb, edge_index, b_curvature, s_curvature,
           eW, eb, bW, bb, sW, sb):
    n, ed = e_emb.shape
    bd = b_emb.shape[1]
    sd = s_emb.shape[1]
    e_edges = edge_index.shape[1]
    per_t = e_edges // _NS
    cb = 125
    nch = per_t // cb
    src3 = edge_index[0].reshape(_NS, nch, cb)
    dst3 = edge_index[1].reshape(_NS, nch, cb)
    c2 = b_curvature.reshape(1, 1)

    def wlayer(l):
        return (eW[l].T, eb[l].reshape(1, ed), bW[l].T, bb[l].reshape(1, bd),
                sW[l].T, sb[l].reshape(1, sd))

    hlo0, hhi0 = _tc_pre(c2, e_emb, b_emb, s_emb, wlayer(0))
    plo0, phi0, d0 = _sc_agg(hlo0, hhi0, src3, dst3, with_deg=True)
    hlo1, hhi1 = _tc_mid(c2, plo0, phi0, d0, wlayer(1), n, ed, bd, sd)
    plo1, phi1 = _sc_agg(hlo1, hhi1, src3, dst3, with_deg=False)
    e2, b2, s2 = _tc_post(c2, plo1, phi1, d0, n, ed, bd, sd)
    return (e2, b2, s2)


# speedup vs baseline: 14.2469x; 1.1536x over previous
import jax
import jax.numpy as jnp
from jax import lax
from jax.experimental import pallas as pl
from jax.experimental.pallas import tpu as pltpu
from jax.experimental.pallas import tpu_sc as plsc

_NC = 2
_NS = 16
_HF = 64
_DW = 16
_R = 2000
_ZR = 128
_NP = 10240


def _leaky(x):
    return jnp.where(x >= 0.0, x, 0.2 * x)


def _l2n(x):
    n = jnp.sqrt(jnp.sum(x * x, axis=-1, keepdims=True))
    return x / jnp.maximum(n, 1e-12)


def _logmap0(b, sc):
    n = jnp.maximum(jnp.sqrt(jnp.sum(b * b, axis=-1, keepdims=True)), 1e-10)
    z = sc * n
    atanh = 0.5 * jnp.log((1.0 + z) / (1.0 - z))
    return (2.0 / sc) * atanh * b / n


def _expmap0(v, sc):
    n = jnp.maximum(jnp.sqrt(jnp.sum(v * v, axis=-1, keepdims=True)), 1e-10)
    return jnp.tanh(sc * n * 0.5) * v / (sc * n)


def _fuse_pre(e, b, s, ewt, eb1, bwt, bb1, swt, sb1, sc):
    he = jnp.dot(e, ewt, preferred_element_type=jnp.float32) + eb1
    hb = jnp.dot(_logmap0(b, sc), bwt, preferred_element_type=jnp.float32) + bb1
    hs = _l2n(jnp.dot(_l2n(s), swt, preferred_element_type=jnp.float32) + sb1)
    return he, jnp.concatenate([hb, hs], axis=1)


def _agg_split(plo_ref, phi_ref, d_ref, sc):
    deg = jnp.maximum(d_ref[...][:, :1], 1.0)
    e1 = _leaky(plo_ref[...] / deg)
    phi = phi_ref[...] / deg
    b1 = _expmap0(phi[:, :32], sc)
    s1 = _l2n(phi[:, 32:])
    return e1, b1, s1


def _wspecs(ed, bd, sd):
    return [
        pl.BlockSpec((ed, ed), lambda i: (0, 0)),
        pl.BlockSpec((1, ed), lambda i: (0, 0)),
        pl.BlockSpec((bd, bd), lambda i: (0, 0)),
        pl.BlockSpec((1, bd), lambda i: (0, 0)),
        pl.BlockSpec((sd, sd), lambda i: (0, 0)),
        pl.BlockSpec((1, sd), lambda i: (0, 0)),
    ]


def _tc_pre(c2, e_emb, b_emb, s_emb, w):
    n, ed = e_emb.shape
    bd = b_emb.shape[1]
    sd = s_emb.shape[1]

    def body(c_ref, e_ref, b_ref, s_ref, ewt, eb1, bwt, bb1, swt, sb1,
             hlo_ref, hhi_ref):
        sc = jnp.sqrt(c_ref[0, 0])
        hlo, hhi = _fuse_pre(e_ref[...], b_ref[...], s_ref[...],
                             ewt[...], eb1[...], bwt[...], bb1[...],
                             swt[...], sb1[...], sc)
        hlo_ref[...] = hlo
        hhi_ref[...] = hhi

    return pl.pallas_call(
        body,
        grid=(n // _R,),
        in_specs=[
            pl.BlockSpec(memory_space=pltpu.SMEM),
            pl.BlockSpec((_R, ed), lambda i: (i, 0)),
            pl.BlockSpec((_R, bd), lambda i: (i, 0)),
            pl.BlockSpec((_R, sd), lambda i: (i, 0)),
        ] + _wspecs(ed, bd, sd),
        out_specs=[
            pl.BlockSpec((_R, _HF), lambda i: (i, 0)),
            pl.BlockSpec((_R, _HF), lambda i: (i, 0)),
        ],
        out_shape=[
            jax.ShapeDtypeStruct((_NP, _HF), jnp.float32),
            jax.ShapeDtypeStruct((_NP, _HF), jnp.float32),
        ],
    )(c2, e_emb, b_emb, s_emb, *w)


def _tc_mid(c2, plo, phi, d, w, n, ed, bd, sd):
    def body(c_ref, plo_ref, phi_ref, d_ref, ewt, eb1, bwt, bb1, swt, sb1,
             hlo_ref, hhi_ref):
        sc = jnp.sqrt(c_ref[0, 0])
        e1, b1, s1 = _agg_split(plo_ref, phi_ref, d_ref, sc)
        hlo, hhi = _fuse_pre(e1, b1, s1, ewt[...], eb1[...], bwt[...],
                             bb1[...], swt[...], sb1[...], sc)
        hlo_ref[...] = hlo
        hhi_ref[...] = hhi

    return pl.pallas_call(
        body,
        grid=(n // _R,),
        in_specs=[
            pl.BlockSpec(memory_space=pltpu.SMEM),
            pl.BlockSpec((_R, _HF), lambda i: (i, 0)),
            pl.BlockSpec((_R, _HF), lambda i: (i, 0)),
            pl.BlockSpec((_R, _DW), lambda i: (i, 0)),
        ] + _wspecs(ed, bd, sd),
        out_specs=[
            pl.BlockSpec((_R, _HF), lambda i: (i, 0)),
            pl.BlockSpec((_R, _HF), lambda i: (i, 0)),
        ],
        out_shape=[
            jax.ShapeDtypeStruct((_NP, _HF), jnp.float32),
            jax.ShapeDtypeStruct((_NP, _HF), jnp.float32),
        ],
    )(c2, plo, phi, d, *w)


def _tc_post(c2, plo, phi, d, n, ed, bd, sd):
    def body(c_ref, plo_ref, phi_ref, d_ref, e_ref, b_ref, s_ref):
        sc = jnp.sqrt(c_ref[0, 0])
        e1, b1, s1 = _agg_split(plo_ref, phi_ref, d_ref, sc)
        e_ref[...] = e1
        b_ref[...] = b1
        s_ref[...] = s1

    return pl.pallas_call(
        body,
        grid=(n // _R,),
        in_specs=[
            pl.BlockSpec(memory_space=pltpu.SMEM),
            pl.BlockSpec((_R, _HF), lambda i: (i, 0)),
            pl.BlockSpec((_R, _HF), lambda i: (i, 0)),
            pl.BlockSpec((_R, _DW), lambda i: (i, 0)),
        ],
        out_specs=[
            pl.BlockSpec((_R, ed), lambda i: (i, 0)),
            pl.BlockSpec((_R, bd), lambda i: (i, 0)),
            pl.BlockSpec((_R, sd), lambda i: (i, 0)),
        ],
        out_shape=[
            jax.ShapeDtypeStruct((n, ed), jnp.float32),
            jax.ShapeDtypeStruct((n, bd), jnp.float32),
            jax.ShapeDtypeStruct((n, sd), jnp.float32),
        ],
    )(c2, plo, phi, d)


def _sc_agg(hlo, hhi, src3, dst3, z64, z16, ones_in, with_deg):
    n = hlo.shape[0]
    nch, cb = src3.shape[1], src3.shape[2]
    rpt = n // _NS

    mesh = plsc.VectorSubcoreMesh(core_axis_name="c", subcore_axis_name="s",
                                  num_cores=_NC, num_subcores=_NS)
    out_type = [jax.ShapeDtypeStruct((n, _HF), jnp.float32),
                jax.ShapeDtypeStruct((n, _HF), jnp.float32)]
    scratch = [
        pltpu.VMEM((nch, cb), jnp.int32),
        pltpu.VMEM((nch, cb), jnp.int32),
        pltpu.VMEM((4, cb, _HF), jnp.float32),
        pltpu.VMEM_SHARED((n, _HF), jnp.float32),
        pltpu.SemaphoreType.DMA,
        pltpu.SemaphoreType.DMA,
        pltpu.SemaphoreType.DMA,
        pltpu.SemaphoreType.DMA,
    ]
    if with_deg:
        out_type.append(jax.ShapeDtypeStruct((n, _DW), jnp.float32))
        scratch += [
            pltpu.VMEM((cb, _DW), jnp.float32),
            pltpu.VMEM_SHARED((n, _DW), jnp.float32),
        ]

    def body(hlo_hbm, hhi_hbm, src_hbm, dst_hbm, z64_hbm, *rest):
        if with_deg:
            z16_hbm, ones_hbm, plo_out, phi_out, d_out, srcv, dstv, rows, \
                acc, gs0, gs1, gs2, gs3, ones, dacc = rest
        else:
            plo_out, phi_out, srcv, dstv, rows, acc, gs0, gs1, gs2, gs3 = rest
        gsems = (gs0, gs1, gs2, gs3)
        cid = lax.axis_index("c")
        sid = lax.axis_index("s")
        base = sid * rpt

        pltpu.sync_copy(z64_hbm, acc.at[pl.ds(base, rpt)])
        if with_deg:
            pltpu.sync_copy(z16_hbm, dacc.at[pl.ds(base, rpt)])
            pltpu.sync_copy(ones_hbm, ones)
        pltpu.sync_copy(src_hbm.at[sid], srcv)
        pltpu.sync_copy(dst_hbm.at[sid], dstv)
        plsc.subcore_barrier()

        def _gather(j, b, sem):
            @pl.when(cid == 0)
            def _():
                pltpu.async_copy(hlo_hbm.at[srcv.at[j]], rows.at[b], sem)

            @pl.when(cid == 1)
            def _():
                pltpu.async_copy(hhi_hbm.at[srcv.at[j]], rows.at[b], sem)

        def _gwait(j, b, sem):
            pltpu.make_async_copy(hlo_hbm.at[srcv.at[j]], rows.at[b], sem).wait()

        def _scat(j, b):
            pltpu.sync_copy(rows.at[b], acc.at[dstv.at[j]], add=True)
            if with_deg:
                @pl.when(cid == 1)
                def _():
                    pltpu.sync_copy(ones, dacc.at[dstv.at[j]], add=True)

        nloop = nch // 4
        for b in range(3):
            _gather(b, b, gsems[b])

        def _quad(g, carry):
            for b in range(4):
                j = 4 * g + b
                bp = (b + 3) % 4

                @pl.when(j + 3 < nch)
                def _():
                    _gather(j + 3, bp, gsems[bp])
                _gwait(j, b, gsems[b])
                _scat(j, b)
            return carry
        lax.fori_loop(0, nloop, _quad, 0)
        plsc.subcore_barrier()

        sl = pl.ds(base, rpt)

        @pl.when(cid == 0)
        def _():
            pltpu.sync_copy(acc.at[sl], plo_out.at[sl])

        @pl.when(cid == 1)
        def _():
            pltpu.sync_copy(acc.at[sl], phi_out.at[sl])
        if with_deg:
            @pl.when(cid == 1)
            def _():
                pltpu.sync_copy(dacc.at[sl], d_out.at[sl])

    f = pl.kernel(body, out_type=out_type, mesh=mesh, scratch_types=scratch,
                  compiler_params=pltpu.CompilerParams(use_tc_tiling_on_sc=False))
    if with_deg:
        return f(hlo, hhi, src3, dst3, z64, z16, ones_in)
    return f(hlo, hhi, src3, dst3, z64)


def kernel(e_emb, b_emb, s_emb, edge_index, b_curvature, s_curvature,
           eW, eb, bW, bb, sW, sb):
    n, ed = e_emb.shape
    bd = b_emb.shape[1]
    sd = s_emb.shape[1]
    e_edges = edge_index.shape[1]
    per_t = e_edges // _NS
    cb = 125
    nch = per_t // cb
    src3 = edge_index[0].reshape(_NS, nch, cb)
    dst3 = edge_index[1].reshape(_NS, nch, cb)
    c2 = b_curvature.reshape(1, 1)

    def wlayer(l):
        return (eW[l].T, eb[l].reshape(1, ed), bW[l].T, bb[l].reshape(1, bd),
                sW[l].T, sb[l].reshape(1, sd))

    rpt = _NP // _NS
    z64 = jnp.zeros((rpt, _HF), jnp.float32)
    z16 = jnp.zeros((rpt, _DW), jnp.float32)
    ones_in = jnp.ones((cb, _DW), jnp.float32)

    hlo0, hhi0 = _tc_pre(c2, e_emb, b_emb, s_emb, wlayer(0))
    plo0, phi0, d0 = _sc_agg(hlo0, hhi0, src3, dst3, z64, z16, ones_in,
                             with_deg=True)
    hlo1, hhi1 = _tc_mid(c2, plo0, phi0, d0, wlayer(1), n, ed, bd, sd)
    plo1, phi1 = _sc_agg(hlo1, hhi1, src3, dst3, z64, z16, ones_in,
                         with_deg=False)
    e2, b2, s2 = _tc_post(c2, plo1, phi1, d0, n, ed, bd, sd)
    return (e2, b2, s2)
